# baseline (speedup 1.0000x reference)
import jax
import jax.numpy as jnp
from jax import lax
from jax.experimental import pallas as pl
from jax.experimental.pallas import tpu as pltpu
from jax.experimental.pallas import tpu_sc as plsc

B = 16384
D = 64
NNEG = 20
NSLOT = NNEG + 1
NW = 32
ROWS_PER_W = B // NW
GR = 128
NCH = ROWS_PER_W * NSLOT // GR
WPW = ROWS_PER_W * NSLOT
GRP = 4
NGRP = NCH // GRP
GROWS = GRP * GR


def _sc_body(pos_t, cidx_hbm, temb, cemb, out,
             tidx, cidx, tbuf, bufs, tsem, *gsem):
    wid = lax.axis_index("s") * 2 + lax.axis_index("c")
    base = wid * ROWS_PER_W

    pltpu.sync_copy(pos_t.at[pl.ds(base, ROWS_PER_W)], tidx)
    pltpu.sync_copy(cidx_hbm.at[pl.ds(wid * NCH, NCH)], cidx)

    def issue(chunk, slot):
        pltpu.async_copy(cemb.at[cidx.at[chunk]],
                         bufs.at[pl.ds(slot * GR, GR)], gsem[slot])

    def wait(slot):
        pltpu.make_async_copy(cemb.at[pl.ds(0, GR)],
                              bufs.at[pl.ds(slot * GR, GR)], gsem[slot]).wait()

    for s in range(2 * GRP):
        issue(s, s)
    for k in range(4):
        pltpu.async_copy(temb.at[tidx.at[pl.ds(k * GR, GR)]],
                         tbuf.at[pl.ds(k * GR, GR)], tsem)
    for k in range(4):
        pltpu.make_async_copy(temb.at[pl.ds(0, GR)],
                              tbuf.at[pl.ds(k * GR, GR)], tsem).wait()
    pltpu.sync_copy(tbuf, out.at[pl.ds(B * NSLOT + base, ROWS_PER_W)])

    def store_group(g, half):
        pltpu.sync_copy(bufs.at[pl.ds(half * GROWS, GROWS)],
                        out.at[pl.ds(wid * WPW + g * GROWS, GROWS)])

    def lap(gg, c0):
        ge = gg * 2
        for c in range(GRP):
            wait(c)
        store_group(ge, 0)
        for c in range(GRP):
            issue((ge + 2) * GRP + c, c)
        go = ge + 1
        for c in range(GRP):
            wait(GRP + c)
        store_group(go, 1)
        for c in range(GRP):
            issue((go + 2) * GRP + c, GRP + c)
        return c0
    lax.fori_loop(0, (NGRP - 3) // 2, lap, jnp.int32(0))

    for c in range(GRP):
        wait(c)
    store_group(NGRP - 3, 0)
    for c in range(GRP):
        issue((NGRP - 1) * GRP + c, c)
    for c in range(GRP):
        wait(GRP + c)
    store_group(NGRP - 2, 1)
    for c in range(GRP):
        wait(c)
    store_group(NGRP - 1, 0)


_sc_gather = pl.kernel(
    _sc_body,
    out_type=jax.ShapeDtypeStruct((B * NSLOT + B, D), jnp.float32),
    mesh=plsc.VectorSubcoreMesh(core_axis_name="c", subcore_axis_name="s"),
    compiler_params=pltpu.CompilerParams(needs_layout_passes=False,
                                         use_tc_tiling_on_sc=False),
    scratch_types=[
        pltpu.VMEM((ROWS_PER_W,), jnp.int32),
        pltpu.VMEM((NCH, GR), jnp.int32),
        pltpu.VMEM((ROWS_PER_W, D), jnp.float32),
        pltpu.VMEM((2 * GROWS, D), jnp.float32),
    ] + [pltpu.SemaphoreType.DMA] * (1 + 2 * GRP),
)


BS = 256
CBLK = BS * NSLOT


def _tc_loss_body(c_ref, t_ref, o_ref):
    ctx = c_ref[:].reshape(BS, NSLOT, D)
    t = t_ref[:]
    dots = jnp.sum(ctx * t[:, None, :], axis=2)
    xc = jnp.clip(dots, -10.0, 10.0)
    slot = lax.broadcasted_iota(jnp.int32, (BS, NSLOT), 1)
    contrib = jnp.where(slot == 0,
                        jnp.log1p(jnp.exp(-xc)),
                        jnp.log1p(jnp.exp(xc)))
    s = jnp.sum(contrib) * (1.0 / B)

    @pl.when(pl.program_id(0) == 0)
    def _():
        o_ref[0, 0] = s

    @pl.when(pl.program_id(0) != 0)
    def _():
        o_ref[0, 0] += s


_tc_loss = pl.pallas_call(
    _tc_loss_body,
    grid=(B // BS,),
    out_shape=jax.ShapeDtypeStruct((1, 1), jnp.float32),
    in_specs=[
        pl.BlockSpec((CBLK, D), lambda i: (i, 0)),
        pl.BlockSpec((BS, D), lambda i: (B * NSLOT // BS + i, 0)),
    ],
    out_specs=pl.BlockSpec((1, 1), lambda i: (0, 0),
                           memory_space=pltpu.SMEM),
)


def kernel(pos_target, pos_context, neg_context, target_emb, context_emb):
    cidx = jnp.concatenate([pos_context[:, None], neg_context], axis=1)
    cidx_hbm = cidx.reshape(B * NSLOT // GR, GR)
    gathered = _sc_gather(pos_target, cidx_hbm, target_emb, context_emb)
    loss = _tc_loss(gathered, gathered)
    return loss[0, 0]


# speedup vs baseline: 1.5339x; 1.5339x over previous
import jax
import jax.numpy as jnp
from jax import lax
from jax.experimental import pallas as pl
from jax.experimental.pallas import tpu as pltpu
from jax.experimental.pallas import tpu_sc as plsc

B = 16384
D = 64
NNEG = 20
NW = 32
ROWS_PER_W = B // NW
CB = 32
NCH = ROWS_PER_W // CB
NSTR = CB * NNEG // 128
NIDX_ROWS = ROWS_PER_W * NNEG // 128
OUTW = 32


def _sc_body(pos_t, pos_c, neg2d, temb, cemb, dots,
             tidx, cidx, nidx, tgtv, ctxv, negv, outv, sem0, sem1):
    wid = lax.axis_index("s") * 2 + lax.axis_index("c")
    base = wid * ROWS_PER_W
    lane = lax.iota(jnp.int32, 16)
    sems = (sem0, sem1)

    pltpu.sync_copy(pos_t.at[pl.ds(base, ROWS_PER_W)], tidx)
    pltpu.sync_copy(pos_c.at[pl.ds(base, ROWS_PER_W)], cidx)
    pltpu.sync_copy(neg2d.at[pl.ds(wid * NIDX_ROWS, NIDX_ROWS)], nidx)

    def issue(ch, b):
        pltpu.async_copy(temb.at[tidx.at[pl.ds(ch * CB, CB)]],
                         tgtv.at[pl.ds(b * CB, CB)], sems[b])
        pltpu.async_copy(cemb.at[cidx.at[pl.ds(ch * CB, CB)]],
                         ctxv.at[pl.ds(b * CB, CB)], sems[b])
        for j in range(NSTR):
            pltpu.async_copy(cemb.at[nidx.at[ch * NSTR + j]],
                             negv.at[pl.ds((b * NSTR + j) * 128, 128)],
                             sems[b])

    def drain(b):
        pltpu.make_async_copy(temb.at[pl.ds(0, CB)],
                              tgtv.at[pl.ds(b * CB, CB)], sems[b]).wait()
        pltpu.make_async_copy(cemb.at[pl.ds(0, CB)],
                              ctxv.at[pl.ds(b * CB, CB)], sems[b]).wait()
        for j in range(NSTR):
            pltpu.make_async_copy(
                cemb.at[pl.ds(0, 128)],
                negv.at[pl.ds((b * NSTR + j) * 128, 128)], sems[b]).wait()

    def compute(ch, b):
        def row_body(r2, c2):
            for u in range(2):
                r = r2 * 2 + u
                rb = b * CB + r
                t0 = tgtv[rb, pl.ds(0, 16)]
                t1 = tgtv[rb, pl.ds(16, 16)]
                t2 = tgtv[rb, pl.ds(32, 16)]
                t3 = tgtv[rb, pl.ds(48, 16)]
                p = ((t0 * ctxv[rb, pl.ds(0, 16)]
                      + t1 * ctxv[rb, pl.ds(16, 16)])
                     + (t2 * ctxv[rb, pl.ds(32, 16)]
                        + t3 * ctxv[rb, pl.ds(48, 16)]))
                rv0 = jnp.where(lane == 0, jnp.sum(p), 0.0)
                rv1 = jnp.zeros((16,), jnp.float32)
                rn = b * CB * NNEG + r * NNEG
                for n in range(NNEG):
                    v = ((t0 * negv[rn + n, pl.ds(0, 16)]
                          + t1 * negv[rn + n, pl.ds(16, 16)])
                         + (t2 * negv[rn + n, pl.ds(32, 16)]
                            + t3 * negv[rn + n, pl.ds(48, 16)]))
                    s = jnp.sum(v)
                    if n < 15:
                        rv0 = jnp.where(lane == 1 + n, s, rv0)
                    else:
                        rv1 = jnp.where(lane == n - 15, s, rv1)
                outv[r, pl.ds(0, 16)] = rv0
                outv[r, pl.ds(16, 16)] = rv1
            return c2
        lax.fori_loop(0, CB // 2, row_body, 0)
        pltpu.sync_copy(outv, dots.at[pl.ds(base + ch * CB, CB)])

    issue(0, 0)
    issue(1, 1)

    def pair_body(i, carry):
        ch0 = i * 2
        for b in range(2):
            drain(b)
            compute(ch0 + b, b)
            issue(ch0 + b + 2, b)
        return carry
    lax.fori_loop(0, NCH // 2 - 1, pair_body, 0)
    for b in range(2):
        drain(b)
        compute(NCH - 2 + b, b)


_sc_dots = pl.kernel(
    _sc_body,
    out_type=jax.ShapeDtypeStruct((B, OUTW), jnp.float32),
    mesh=plsc.VectorSubcoreMesh(core_axis_name="c", subcore_axis_name="s"),
    compiler_params=pltpu.CompilerParams(needs_layout_passes=False,
                                         use_tc_tiling_on_sc=False),
    scratch_types=[
        pltpu.VMEM((ROWS_PER_W,), jnp.int32),
        pltpu.VMEM((ROWS_PER_W,), jnp.int32),
        pltpu.VMEM((NIDX_ROWS, 128), jnp.int32),
        pltpu.VMEM((2 * CB, D), jnp.float32),
        pltpu.VMEM((2 * CB, D), jnp.float32),
        pltpu.VMEM((2 * CB * NNEG, D), jnp.float32),
        pltpu.VMEM((CB, OUTW), jnp.float32),
        pltpu.SemaphoreType.DMA,
        pltpu.SemaphoreType.DMA,
    ],
)


def _tc_loss_body(d_ref, o_ref):
    x = d_ref[:]
    col = lax.broadcasted_iota(jnp.int32, x.shape, 1) % OUTW
    xc = jnp.clip(x, -10.0, 10.0)
    pos_f = jnp.log1p(jnp.exp(-xc))
    neg_f = jnp.log1p(jnp.exp(xc))
    contrib = jnp.where(col == 0, pos_f,
                        jnp.where(col <= NNEG, neg_f, 0.0))
    o_ref[0, 0] = jnp.sum(contrib) * (1.0 / B)


_tc_loss = pl.pallas_call(
    _tc_loss_body,
    out_shape=jax.ShapeDtypeStruct((1, 1), jnp.float32),
    in_specs=[pl.BlockSpec(memory_space=pltpu.VMEM)],
    out_specs=pl.BlockSpec(memory_space=pltpu.SMEM),
)


def kernel(pos_target, pos_context, neg_context, target_emb, context_emb):
    neg2d = neg_context.reshape(B * NNEG // 128, 128)
    dots = _sc_dots(pos_target, pos_context, neg2d, target_emb, context_emb)
    loss = _tc_loss(dots.reshape(B * OUTW // 128, 128))
    return loss[0, 0]


# speedup vs baseline: 1.5364x; 1.0016x over previous
import jax
import jax.numpy as jnp
from jax import lax
from jax.experimental import pallas as pl
from jax.experimental.pallas import tpu as pltpu
from jax.experimental.pallas import tpu_sc as plsc

B = 16384
D = 64
NNEG = 20
NW = 32
ROWS_PER_W = B // NW
CB = 32
NCH = ROWS_PER_W // CB
NSTR = CB * NNEG // 128
NIDX_ROWS = ROWS_PER_W * NNEG // 128
OUTW = 32


def _sc_body(pos_t, pos_c, neg2d, temb, cemb, dots,
             tidx, cidx, nidx, tgtv, ctxv, negv, outv, sem0, sem1):
    wid = lax.axis_index("s") * 2 + lax.axis_index("c")
    base = wid * ROWS_PER_W
    lane = lax.iota(jnp.int32, 16)
    sems = (sem0, sem1)

    pltpu.sync_copy(pos_t.at[pl.ds(base, ROWS_PER_W)], tidx)
    pltpu.sync_copy(pos_c.at[pl.ds(base, ROWS_PER_W)], cidx)
    pltpu.sync_copy(neg2d.at[pl.ds(wid * NIDX_ROWS, NIDX_ROWS)], nidx)

    def issue(ch, b):
        pltpu.async_copy(temb.at[tidx.at[pl.ds(ch * CB, CB)]],
                         tgtv.at[pl.ds(b * CB, CB)], sems[b])
        pltpu.async_copy(cemb.at[cidx.at[pl.ds(ch * CB, CB)]],
                         ctxv.at[pl.ds(b * CB, CB)], sems[b])
        for j in range(NSTR):
            pltpu.async_copy(cemb.at[nidx.at[ch * NSTR + j]],
                             negv.at[pl.ds((b * NSTR + j) * 128, 128)],
                             sems[b])

    def drain(b):
        pltpu.make_async_copy(temb.at[pl.ds(0, CB)],
                              tgtv.at[pl.ds(b * CB, CB)], sems[b]).wait()
        pltpu.make_async_copy(cemb.at[pl.ds(0, CB)],
                              ctxv.at[pl.ds(b * CB, CB)], sems[b]).wait()
        for j in range(NSTR):
            pltpu.make_async_copy(
                cemb.at[pl.ds(0, 128)],
                negv.at[pl.ds((b * NSTR + j) * 128, 128)], sems[b]).wait()

    def compute(ch, b):
        def row_body(r, c2):
            rb = b * CB + r
            t0 = tgtv[rb, pl.ds(0, 16)]
            t1 = tgtv[rb, pl.ds(16, 16)]
            t2 = tgtv[rb, pl.ds(32, 16)]
            t3 = tgtv[rb, pl.ds(48, 16)]
            p = (t0 * ctxv[rb, pl.ds(0, 16)] + t1 * ctxv[rb, pl.ds(16, 16)]
                 + t2 * ctxv[rb, pl.ds(32, 16)]
                 + t3 * ctxv[rb, pl.ds(48, 16)])
            rv0 = jnp.where(lane == 0, jnp.sum(p), 0.0)
            rv1 = jnp.zeros((16,), jnp.float32)
            rn = b * CB * NNEG + r * NNEG
            for n in range(NNEG):
                v = (t0 * negv[rn + n, pl.ds(0, 16)]
                     + t1 * negv[rn + n, pl.ds(16, 16)]
                     + t2 * negv[rn + n, pl.ds(32, 16)]
                     + t3 * negv[rn + n, pl.ds(48, 16)])
                s = jnp.sum(v)
                if n < 15:
                    rv0 = jnp.where(lane == 1 + n, s, rv0)
                else:
                    rv1 = jnp.where(lane == n - 15, s, rv1)
            outv[r, pl.ds(0, 16)] = rv0
            outv[r, pl.ds(16, 16)] = rv1
            return c2
        lax.fori_loop(0, CB, row_body, 0)
        pltpu.sync_copy(outv, dots.at[pl.ds(base + ch * CB, CB)])

    issue(0, 0)
    issue(1, 1)

    def pair_body(i, carry):
        ch0 = i * 2
        for b in range(2):
            drain(b)
            compute(ch0 + b, b)
            issue(ch0 + b + 2, b)
        return carry
    lax.fori_loop(0, NCH // 2 - 1, pair_body, 0)
    for b in range(2):
        drain(b)
        compute(NCH - 2 + b, b)


_sc_dots = pl.kernel(
    _sc_body,
    out_type=jax.ShapeDtypeStruct((B, OUTW), jnp.float32),
    mesh=plsc.VectorSubcoreMesh(core_axis_name="c", subcore_axis_name="s"),
    compiler_params=pltpu.CompilerParams(needs_layout_passes=False,
                                         use_tc_tiling_on_sc=False),
    scratch_types=[
        pltpu.VMEM((ROWS_PER_W,), jnp.int32),
        pltpu.VMEM((ROWS_PER_W,), jnp.int32),
        pltpu.VMEM((NIDX_ROWS, 128), jnp.int32),
        pltpu.VMEM((2 * CB, D), jnp.float32),
        pltpu.VMEM((2 * CB, D), jnp.float32),
        pltpu.VMEM((2 * CB * NNEG, D), jnp.float32),
        pltpu.VMEM((CB, OUTW), jnp.float32),
        pltpu.SemaphoreType.DMA,
        pltpu.SemaphoreType.DMA,
    ],
)


def _tc_loss_body(d_ref, o_ref):
    x = d_ref[:]
    col = lax.broadcasted_iota(jnp.int32, x.shape, 1) % OUTW
    xc = jnp.clip(x, -10.0, 10.0)
    pos_f = jnp.log1p(jnp.exp(-xc))
    neg_f = jnp.log1p(jnp.exp(xc))
    contrib = jnp.where(col == 0, pos_f,
                        jnp.where(col <= NNEG, neg_f, 0.0))
    o_ref[0, 0] = jnp.sum(contrib) * (1.0 / B)


_tc_loss = pl.pallas_call(
    _tc_loss_body,
    out_shape=jax.ShapeDtypeStruct((1, 1), jnp.float32),
    in_specs=[pl.BlockSpec(memory_space=pltpu.VMEM)],
    out_specs=pl.BlockSpec(memory_space=pltpu.SMEM),
)


def kernel(pos_target, pos_context, neg_context, target_emb, context_emb):
    neg2d = neg_context.reshape(B * NNEG // 128, 128)
    dots = _sc_dots(pos_target, pos_context, neg2d, target_emb, context_emb)
    loss = _tc_loss(dots.reshape(B * OUTW // 128, 128))
    return loss[0, 0]
